# baseline (speedup 1.0000x reference)
import functools

import jax
import jax.numpy as jnp
from jax import lax
from jax.experimental import pallas as pl
from jax.experimental.pallas import tpu as pltpu
from jax.experimental.pallas import tpu_sc as plsc

D = 128
DW = D // 2
E = 320000
NC, NS, L = 2, 16, 16
NW = NC * NS
PER_W = E // NW
C = 80
NCHUNK = PER_W // C
NPAIR = (NCHUNK - 1) // 2


NB = 2


def _sc_kernel(x1_hbm, x2_hbm, i_hbm, out_hbm, *scratch):
  idx1_v, idx2_v = scratch[0], scratch[1]
  rb1 = scratch[2:2 + NB]
  rb2 = scratch[2 + NB:2 + 2 * NB]
  outw_v = scratch[3 + 2 * NB]
  si1, si2 = scratch[4 + 2 * NB], scratch[5 + 2 * NB]
  s1 = scratch[6 + 2 * NB:6 + 3 * NB]
  s2 = scratch[6 + 3 * NB:6 + 4 * NB]

  wid = lax.axis_index("s") * NC + lax.axis_index("c")
  wbase = wid * PER_W
  lane = lax.iota(jnp.int32, L)

  cpi1 = pltpu.async_copy(i_hbm.at[0, pl.ds(wbase, PER_W)], idx1_v, si1)
  cpi2 = pltpu.async_copy(i_hbm.at[1, pl.ds(wbase, PER_W)], idx2_v, si2)
  cpi1.wait()
  cpi2.wait()

  def start(c, b):
    pltpu.async_copy(x1_hbm.at[idx1_v.at[pl.ds(c * C, C)]], rb1[b], s1[b])
    pltpu.async_copy(x2_hbm.at[idx2_v.at[pl.ds(c * C, C)]], rb2[b], s2[b])

  def wait(c, b):
    pltpu.make_async_copy(
        x1_hbm.at[idx1_v.at[pl.ds(c * C, C)]], rb1[b], s1[b]).wait()
    pltpu.make_async_copy(
        x2_hbm.at[idx2_v.at[pl.ds(c * C, C)]], rb2[b], s2[b]).wait()

  def compute(c, b):
    r1, r2 = rb1[b], rb2[b]

    def group_body(g, carry):
      row = g * L + lane

      def j_body(j, accs):
        colj = (lane + j) & (L - 1)
        out = list(accs)
        for k in range(DW // L):
          col = colj + k * L
          w1 = plsc.load_gather(r1, [row, col])
          w2 = plsc.load_gather(r2, [row, col])
          p = plsc.bitcast(w1, jnp.bfloat16) * plsc.bitcast(w2, jnp.bfloat16)
          p0, p1 = plsc.unpack(p, format=plsc.PackFormat.INTERLEAVED)
          out[2 * k] = out[2 * k] + p0
          out[2 * k + 1] = out[2 * k + 1] + p1
        return tuple(out)

      accs = list(lax.fori_loop(
          0, L, j_body,
          tuple(jnp.zeros((L,), jnp.float32) for _ in range(8)),
          unroll=8))
      while len(accs) > 1:
        accs = [a + b for a, b in zip(accs[::2], accs[1::2])]
      outw_v[pl.ds(c * C + g * L, L)] = accs[0]
      return carry
    lax.fori_loop(0, C // L, group_body, 0)

  start(0, 0)

  def pair_body(i, carry):
    c0 = 2 * i
    start(c0 + 1, 1)
    wait(c0, 0)
    compute(c0, 0)
    start(c0 + 2, 0)
    wait(c0 + 1, 1)
    compute(c0 + 1, 1)
    return carry

  lax.fori_loop(0, NPAIR, pair_body, 0)
  wait(NCHUNK - 1, 0)
  compute(NCHUNK - 1, 0)

  pltpu.sync_copy(outw_v, out_hbm.at[pl.ds(wbase, PER_W)])


@functools.partial(
    pl.kernel,
    mesh=plsc.VectorSubcoreMesh(core_axis_name="c", subcore_axis_name="s"),
    out_type=jax.ShapeDtypeStruct((E,), jnp.float32),
    compiler_params=pltpu.CompilerParams(needs_layout_passes=False,
                                         use_tc_tiling_on_sc=False),
    scratch_types=(
        [pltpu.VMEM((PER_W,), jnp.int32)] * 2
        + [pltpu.VMEM((C, DW), jnp.int32)] * (2 * NB)
        + [pltpu.VMEM((L * (L + 1),), jnp.float32),
           pltpu.VMEM((PER_W,), jnp.float32)]
        + [pltpu.SemaphoreType.DMA] * (2 + 2 * NB)
    ),
)
def _edge_dot(x1, x2, ei, out, *scratch):
  _sc_kernel(x1, x2, ei, out, *scratch)


def _pack_body(x1_ref, x2_ref, o1_ref, o2_ref):
  for x_ref, o_ref in ((x1_ref, o1_ref), (x2_ref, o2_ref)):
    x = x_ref[...]
    lo = lax.bitcast_convert_type(
        x[:, :DW].astype(jnp.bfloat16), jnp.uint16).astype(jnp.uint32)
    hi = lax.bitcast_convert_type(
        x[:, DW:].astype(jnp.bfloat16), jnp.uint16).astype(jnp.uint32)
    o_ref[...] = lax.bitcast_convert_type(lo | (hi << 16), jnp.int32)


def _pack_tables(x1, x2):
  n = x1.shape[0]
  return pl.pallas_call(
      _pack_body,
      out_shape=(jax.ShapeDtypeStruct((n, DW), jnp.int32),
                 jax.ShapeDtypeStruct((n, DW), jnp.int32)),
  )(x1, x2)


def kernel(x_nt1, x_nt2, edge_label_index):
  x1p, x2p = _pack_tables(x_nt1, x_nt2)
  return _edge_dot(x1p, x2p, edge_label_index.astype(jnp.int32))


# speedup vs baseline: 1.0111x; 1.0111x over previous
import functools

import jax
import jax.numpy as jnp
from jax import lax
from jax.experimental import pallas as pl
from jax.experimental.pallas import tpu as pltpu
from jax.experimental.pallas import tpu_sc as plsc

D = 128
DW = D // 2
E = 320000
NC, NS, L = 2, 16, 16
NW = NC * NS
PER_W = E // NW
C = 80
NCHUNK = PER_W // C
NPAIR = (NCHUNK - 1) // 2


NB = 2


def _sc_kernel(x1_hbm, x2_hbm, i_hbm, out_hbm, *scratch):
  idx1_v, idx2_v = scratch[0], scratch[1]
  rb1 = scratch[2:2 + NB]
  rb2 = scratch[2 + NB:2 + 2 * NB]
  outw_v = scratch[3 + 2 * NB]
  si1, si2 = scratch[4 + 2 * NB], scratch[5 + 2 * NB]
  s1 = scratch[6 + 2 * NB:6 + 3 * NB]
  s2 = scratch[6 + 3 * NB:6 + 4 * NB]

  wid = lax.axis_index("s") * NC + lax.axis_index("c")
  wbase = wid * PER_W
  lane = lax.iota(jnp.int32, L)

  cpi1 = pltpu.async_copy(i_hbm.at[0, pl.ds(wbase, PER_W)], idx1_v, si1)
  cpi2 = pltpu.async_copy(i_hbm.at[1, pl.ds(wbase, PER_W)], idx2_v, si2)
  cpi1.wait()
  cpi2.wait()

  def start(c, b):
    pltpu.async_copy(x1_hbm.at[idx1_v.at[pl.ds(c * C, C)]], rb1[b], s1[b])
    pltpu.async_copy(x2_hbm.at[idx2_v.at[pl.ds(c * C, C)]], rb2[b], s2[b])

  def wait(c, b):
    pltpu.make_async_copy(
        x1_hbm.at[idx1_v.at[pl.ds(c * C, C)]], rb1[b], s1[b]).wait()
    pltpu.make_async_copy(
        x2_hbm.at[idx2_v.at[pl.ds(c * C, C)]], rb2[b], s2[b]).wait()

  def compute(c, b):
    r1, r2 = rb1[b], rb2[b]

    def group_body(g, carry):
      row = g * L + lane

      def j_body(j, accs):
        colj = (lane + j) & (L - 1)
        out = list(accs)
        for k in range(DW // L):
          col = colj + k * L
          w1 = plsc.load_gather(r1, [row, col])
          w2 = plsc.load_gather(r2, [row, col])
          p = plsc.bitcast(w1, jnp.bfloat16) * plsc.bitcast(w2, jnp.bfloat16)
          p0, p1 = plsc.unpack(p, format=plsc.PackFormat.INTERLEAVED)
          out[2 * k] = out[2 * k] + p0
          out[2 * k + 1] = out[2 * k + 1] + p1
        return tuple(out)

      accs = list(lax.fori_loop(
          0, L, j_body,
          tuple(jnp.zeros((L,), jnp.float32) for _ in range(8)),
          unroll=4))
      while len(accs) > 1:
        accs = [a + b for a, b in zip(accs[::2], accs[1::2])]
      outw_v[pl.ds(c * C + g * L, L)] = accs[0]
      return carry
    lax.fori_loop(0, C // L, group_body, 0)

  start(0, 0)

  def pair_body(i, carry):
    c0 = 2 * i
    start(c0 + 1, 1)
    wait(c0, 0)
    compute(c0, 0)
    start(c0 + 2, 0)
    wait(c0 + 1, 1)
    compute(c0 + 1, 1)
    return carry

  lax.fori_loop(0, NPAIR, pair_body, 0)
  wait(NCHUNK - 1, 0)
  compute(NCHUNK - 1, 0)

  pltpu.sync_copy(outw_v, out_hbm.at[pl.ds(wbase, PER_W)])


@functools.partial(
    pl.kernel,
    mesh=plsc.VectorSubcoreMesh(core_axis_name="c", subcore_axis_name="s"),
    out_type=jax.ShapeDtypeStruct((E,), jnp.float32),
    compiler_params=pltpu.CompilerParams(needs_layout_passes=False,
                                         use_tc_tiling_on_sc=False),
    scratch_types=(
        [pltpu.VMEM((PER_W,), jnp.int32)] * 2
        + [pltpu.VMEM((C, DW), jnp.int32)] * (2 * NB)
        + [pltpu.VMEM((L * (L + 1),), jnp.float32),
           pltpu.VMEM((PER_W,), jnp.float32)]
        + [pltpu.SemaphoreType.DMA] * (2 + 2 * NB)
    ),
)
def _edge_dot(x1, x2, ei, out, *scratch):
  _sc_kernel(x1, x2, ei, out, *scratch)


def _pack_body(x1_ref, x2_ref, o1_ref, o2_ref):
  for x_ref, o_ref in ((x1_ref, o1_ref), (x2_ref, o2_ref)):
    x = x_ref[...]
    lo = lax.bitcast_convert_type(
        x[:, :DW].astype(jnp.bfloat16), jnp.uint16).astype(jnp.uint32)
    hi = lax.bitcast_convert_type(
        x[:, DW:].astype(jnp.bfloat16), jnp.uint16).astype(jnp.uint32)
    o_ref[...] = lax.bitcast_convert_type(lo | (hi << 16), jnp.int32)


def _pack_tables(x1, x2):
  n = x1.shape[0]
  return pl.pallas_call(
      _pack_body,
      out_shape=(jax.ShapeDtypeStruct((n, DW), jnp.int32),
                 jax.ShapeDtypeStruct((n, DW), jnp.int32)),
  )(x1, x2)


def kernel(x_nt1, x_nt2, edge_label_index):
  x1p, x2p = _pack_tables(x_nt1, x_nt2)
  return _edge_dot(x1p, x2p, edge_label_index.astype(jnp.int32))


# speedup vs baseline: 1.0120x; 1.0009x over previous
import functools

import jax
import jax.numpy as jnp
from jax import lax
from jax.experimental import pallas as pl
from jax.experimental.pallas import tpu as pltpu
from jax.experimental.pallas import tpu_sc as plsc

D = 128
DW = D // 2
E = 320000
NC, NS, L = 2, 16, 16
NW = NC * NS
PER_W = E // NW
C = 80
NCHUNK = PER_W // C
NPAIR = (NCHUNK - 1) // 2


NB = 2


def _sc_kernel(x1_hbm, x2_hbm, i_hbm, out_hbm, *scratch):
  idx1_v, idx2_v = scratch[0], scratch[1]
  rb1 = scratch[2:2 + NB]
  rb2 = scratch[2 + NB:2 + 2 * NB]
  outw_v = scratch[2 + 2 * NB]
  si1, si2 = scratch[3 + 2 * NB], scratch[4 + 2 * NB]
  s1 = scratch[5 + 2 * NB:5 + 3 * NB]
  s2 = scratch[5 + 3 * NB:5 + 4 * NB]

  wid = lax.axis_index("s") * NC + lax.axis_index("c")
  wbase = wid * PER_W
  lane = lax.iota(jnp.int32, L)

  cpi1 = pltpu.async_copy(i_hbm.at[0, pl.ds(wbase, PER_W)], idx1_v, si1)
  cpi2 = pltpu.async_copy(i_hbm.at[1, pl.ds(wbase, PER_W)], idx2_v, si2)
  cpi1.wait()
  cpi2.wait()

  def start(c, b):
    pltpu.async_copy(x1_hbm.at[idx1_v.at[pl.ds(c * C, C)]], rb1[b], s1[b])
    pltpu.async_copy(x2_hbm.at[idx2_v.at[pl.ds(c * C, C)]], rb2[b], s2[b])

  def wait(c, b):
    pltpu.make_async_copy(
        x1_hbm.at[idx1_v.at[pl.ds(c * C, C)]], rb1[b], s1[b]).wait()
    pltpu.make_async_copy(
        x2_hbm.at[idx2_v.at[pl.ds(c * C, C)]], rb2[b], s2[b]).wait()

  def compute(c, b):
    r1, r2 = rb1[b], rb2[b]

    def group_body(g, carry):
      row = g * L + lane

      def j_body(j, accs):
        colj = (lane + j) & (L - 1)
        out = list(accs)
        for k in range(DW // L):
          col = colj + k * L
          w1 = plsc.load_gather(r1, [row, col])
          w2 = plsc.load_gather(r2, [row, col])
          p = plsc.bitcast(w1, jnp.bfloat16) * plsc.bitcast(w2, jnp.bfloat16)
          p0, p1 = plsc.unpack(p, format=plsc.PackFormat.INTERLEAVED)
          out[2 * k] = out[2 * k] + p0
          out[2 * k + 1] = out[2 * k + 1] + p1
        return tuple(out)

      accs = list(lax.fori_loop(
          0, L, j_body,
          tuple(jnp.zeros((L,), jnp.float32) for _ in range(8)),
          unroll=4))
      while len(accs) > 1:
        accs = [a + b for a, b in zip(accs[::2], accs[1::2])]
      outw_v[pl.ds(c * C + g * L, L)] = accs[0]
      return carry
    lax.fori_loop(0, C // L, group_body, 0)

  start(0, 0)

  def pair_body(i, carry):
    c0 = 2 * i
    start(c0 + 1, 1)
    wait(c0, 0)
    compute(c0, 0)
    start(c0 + 2, 0)
    wait(c0 + 1, 1)
    compute(c0 + 1, 1)
    return carry

  lax.fori_loop(0, NPAIR, pair_body, 0)
  wait(NCHUNK - 1, 0)
  compute(NCHUNK - 1, 0)

  pltpu.sync_copy(outw_v, out_hbm.at[pl.ds(wbase, PER_W)])


@functools.partial(
    pl.kernel,
    mesh=plsc.VectorSubcoreMesh(core_axis_name="c", subcore_axis_name="s"),
    out_type=jax.ShapeDtypeStruct((E,), jnp.float32),
    compiler_params=pltpu.CompilerParams(needs_layout_passes=False,
                                         use_tc_tiling_on_sc=False),
    scratch_types=(
        [pltpu.VMEM((PER_W,), jnp.int32)] * 2
        + [pltpu.VMEM((C, DW), jnp.int32)] * (2 * NB)
        + [pltpu.VMEM((PER_W,), jnp.float32)]
        + [pltpu.SemaphoreType.DMA] * (2 + 2 * NB)
    ),
)
def _edge_dot(x1, x2, ei, out, *scratch):
  _sc_kernel(x1, x2, ei, out, *scratch)


def _pack_body(x1_ref, x2_ref, o1_ref, o2_ref):
  for x_ref, o_ref in ((x1_ref, o1_ref), (x2_ref, o2_ref)):
    x = x_ref[...]
    lo = lax.bitcast_convert_type(
        x[:, :DW].astype(jnp.bfloat16), jnp.uint16).astype(jnp.uint32)
    hi = lax.bitcast_convert_type(
        x[:, DW:].astype(jnp.bfloat16), jnp.uint16).astype(jnp.uint32)
    o_ref[...] = lax.bitcast_convert_type(lo | (hi << 16), jnp.int32)


def _pack_tables(x1, x2):
  n = x1.shape[0]
  return pl.pallas_call(
      _pack_body,
      out_shape=(jax.ShapeDtypeStruct((n, DW), jnp.int32),
                 jax.ShapeDtypeStruct((n, DW), jnp.int32)),
  )(x1, x2)


def kernel(x_nt1, x_nt2, edge_label_index):
  x1p, x2p = _pack_tables(x_nt1, x_nt2)
  return _edge_dot(x1p, x2p, edge_label_index.astype(jnp.int32))
